# Initial kernel scaffold; baseline (speedup 1.0000x reference)
import jax
import jax.numpy as jnp
from jax.experimental import pallas as pl


def kernel(input, weight):
    raise NotImplementedError("write your pallas kernel here")


# speedup vs baseline: 1.0930x; 1.0930x over previous
import functools

import jax
import jax.numpy as jnp
from jax import lax
from jax.experimental import pallas as pl
from jax.experimental.pallas import tpu as pltpu
from jax.experimental.pallas import tpu_sc as plsc

VOCAB = 1_000_000
EMBED_DIM = 32
CHUNK = 1024


@functools.lru_cache(maxsize=None)
def _build(batch_flat: int):
    info = plsc.get_sparse_core_info()
    nc, ns = info.num_cores, info.num_subcores
    nw = nc * ns
    assert batch_flat % (nw * CHUNK) == 0
    b_per_w = batch_flat // nw
    n_chunks = b_per_w // CHUNK

    mesh = plsc.VectorSubcoreMesh(core_axis_name="c", subcore_axis_name="s")

    @functools.partial(
        pl.kernel,
        mesh=mesh,
        out_type=jax.ShapeDtypeStruct((batch_flat, EMBED_DIM), jnp.float32),
        scratch_types=[
            pltpu.VMEM((CHUNK,), jnp.int32),
            pltpu.VMEM((CHUNK, EMBED_DIM), jnp.float32),
            pltpu.SemaphoreType.DMA,
        ],
        compiler_params=pltpu.CompilerParams(use_tc_tiling_on_sc=False),
    )
    def grab(table_hbm, idx_hbm, out_hbm, idx_v, rows_v, sem):
        wid = lax.axis_index("s") * nc + lax.axis_index("c")
        base = wid * b_per_w

        def body(i, _):
            off = base + i * CHUNK
            pltpu.sync_copy(idx_hbm.at[pl.ds(off, CHUNK)], idx_v)
            pltpu.async_copy(table_hbm.at[idx_v], rows_v, sem).wait()
            pltpu.sync_copy(rows_v, out_hbm.at[pl.ds(off, CHUNK)])
            return 0

        lax.fori_loop(0, n_chunks, body, 0)

    return grab


def kernel(input, weight):
    b, h = input.shape
    flat_idx = input.reshape(b * h).astype(jnp.int32)
    out = _build(b * h)(weight, flat_idx)
    return out.reshape(b, h, EMBED_DIM)


# speedup vs baseline: 1.1103x; 1.0158x over previous
import functools

import jax
import jax.numpy as jnp
from jax import lax
from jax.experimental import pallas as pl
from jax.experimental.pallas import tpu as pltpu
from jax.experimental.pallas import tpu_sc as plsc

VOCAB = 1_000_000
EMBED_DIM = 32
CHUNK = 512
NBUF = 5


@functools.lru_cache(maxsize=None)
def _build(batch_flat: int):
    info = plsc.get_sparse_core_info()
    nc, ns = info.num_cores, info.num_subcores
    nw = nc * ns
    assert batch_flat % (nw * CHUNK * NBUF) == 0
    b_per_w = batch_flat // nw
    n_chunks = b_per_w // CHUNK
    n_groups = n_chunks // NBUF

    mesh = plsc.VectorSubcoreMesh(core_axis_name="c", subcore_axis_name="s")

    @functools.partial(
        pl.kernel,
        mesh=mesh,
        out_type=jax.ShapeDtypeStruct((batch_flat, EMBED_DIM), jnp.float32),
        scratch_types=[
            pltpu.VMEM((n_chunks, CHUNK), jnp.int32),
            pltpu.VMEM((NBUF, CHUNK, EMBED_DIM), jnp.float32),
            pltpu.SemaphoreType.DMA((NBUF,)),
            pltpu.SemaphoreType.DMA((NBUF,)),
        ],
        compiler_params=pltpu.CompilerParams(use_tc_tiling_on_sc=False),
    )
    def grab(table_hbm, idx_hbm, out_hbm, idx_v, rows_v, gsem, osem):
        wid = lax.axis_index("s") * nc + lax.axis_index("c")
        base = wid * b_per_w
        pltpu.sync_copy(idx_hbm.at[wid], idx_v)

        def gather(i, b):
            return pltpu.make_async_copy(
                table_hbm.at[idx_v.at[i]], rows_v.at[b], gsem.at[b])

        def writeback(i, b):
            return pltpu.make_async_copy(
                rows_v.at[b],
                out_hbm.at[pl.ds(base + i * CHUNK, CHUNK)],
                osem.at[b])

        for b in range(NBUF):
            gather(b, b).start()

        def group(g, _):
            i0 = g * NBUF
            for b in range(NBUF):
                gather(i0 + b, b).wait()
                writeback(i0 + b, b).start()
            for b in range(NBUF):
                writeback(i0 + b, b).wait()
                gather(i0 + NBUF + b, b).start()
            return 0

        lax.fori_loop(0, n_groups - 1, group, 0)

        i0 = (n_groups - 1) * NBUF
        for b in range(NBUF):
            gather(i0 + b, b).wait()
            writeback(i0 + b, b).start()
        for b in range(NBUF):
            writeback(i0 + b, b).wait()

    return grab, nw, n_chunks


def kernel(input, weight):
    b, h = input.shape
    grab, nw, n_chunks = _build(b * h)
    flat_idx = input.reshape(nw, n_chunks, CHUNK).astype(jnp.int32)
    out = grab(weight, flat_idx)
    return out.reshape(b, h, EMBED_DIM)


# speedup vs baseline: 1.7713x; 1.5954x over previous
import functools

import jax
import jax.numpy as jnp
from jax import lax
from jax.experimental import pallas as pl
from jax.experimental.pallas import tpu as pltpu
from jax.experimental.pallas import tpu_sc as plsc

VOCAB = 1_000_000
EMBED_DIM = 32
NBUF = 8


@functools.lru_cache(maxsize=None)
def _build(batch: int, hist: int):
    info = plsc.get_sparse_core_info()
    nc, ns = info.num_cores, info.num_subcores
    nw = nc * ns
    assert batch % (nw * NBUF) == 0
    rows_per_w = batch // nw
    n_groups = rows_per_w // NBUF

    mesh = plsc.VectorSubcoreMesh(core_axis_name="c", subcore_axis_name="s")

    @functools.partial(
        pl.kernel,
        mesh=mesh,
        out_type=jax.ShapeDtypeStruct((batch, hist, EMBED_DIM), jnp.float32),
        scratch_types=[
            pltpu.VMEM((rows_per_w, hist), jnp.int32),
            pltpu.VMEM((NBUF, hist, EMBED_DIM), jnp.float32),
            pltpu.SemaphoreType.DMA((NBUF,)),
            pltpu.SemaphoreType.DMA((NBUF,)),
        ],
        compiler_params=pltpu.CompilerParams(use_tc_tiling_on_sc=False),
    )
    def grab(table_hbm, idx_hbm, out_hbm, idx_v, rows_v, gsem, osem):
        wid = lax.axis_index("s") * nc + lax.axis_index("c")
        row0 = wid * rows_per_w
        pltpu.sync_copy(idx_hbm.at[pl.ds(row0, rows_per_w)], idx_v)

        def gather(r, b):
            return pltpu.make_async_copy(
                table_hbm.at[idx_v.at[r]], rows_v.at[b], gsem.at[b])

        def writeback(r, b):
            return pltpu.make_async_copy(
                rows_v.at[b], out_hbm.at[row0 + r], osem.at[b])

        for b in range(NBUF):
            gather(b, b).start()

        def group(g, _):
            r0 = g * NBUF
            for b in range(NBUF):
                gather(r0 + b, b).wait()
                writeback(r0 + b, b).start()
            for b in range(NBUF):
                writeback(r0 + b, b).wait()
                gather(r0 + NBUF + b, b).start()
            return 0

        lax.fori_loop(0, n_groups - 1, group, 0)

        r0 = (n_groups - 1) * NBUF
        for b in range(NBUF):
            gather(r0 + b, b).wait()
            writeback(r0 + b, b).start()
        for b in range(NBUF):
            writeback(r0 + b, b).wait()

    return grab


def kernel(input, weight):
    b, h = input.shape
    return _build(b, h)(weight, input)
